# Initial kernel scaffold; baseline (speedup 1.0000x reference)
import functools

import jax
import jax.numpy as jnp
from jax import lax
from jax.experimental import pallas as pl
from jax.experimental.pallas import tpu as pltpu
from jax.experimental.pallas import tpu_sc as plsc

_NUM_WORKERS = 32


def _make_gather(v, d, b, ch):
  bpw = b // _NUM_WORKERS
  nch = bpw // ch
  mesh = plsc.VectorSubcoreMesh(core_axis_name="c", subcore_axis_name="s")

  @functools.partial(
      pl.kernel,
      mesh=mesh,
      out_type=jax.ShapeDtypeStruct((b, d), jnp.float32),
      scratch_types=[
          pltpu.VMEM((ch,), jnp.int32),
          pltpu.VMEM((ch, d), jnp.float32),
          pltpu.SemaphoreType.DMA,
      ],
  )
  def gather(table_hbm, idx_hbm, out_hbm, idx_v, rows_v, sem):
    wid = lax.axis_index("s") * 2 + lax.axis_index("c")
    w_base = wid * bpw

    def body(j, carry):
      base = w_base + j * ch
      pltpu.sync_copy(idx_hbm.at[pl.ds(base, ch)], idx_v)
      pltpu.async_copy(table_hbm.at[idx_v], rows_v, sem).wait()
      pltpu.sync_copy(rows_v, out_hbm.at[pl.ds(base, ch)])
      return carry

    lax.fori_loop(0, nch, body, 0)

  return gather


def kernel(x, pe):
  b, l = x.shape
  v = pe.shape[0]
  d = pe.shape[-1]
  total = b * l
  idx = x.reshape(total)
  table = pe.reshape(v, d)
  out = _make_gather(v, d, total, ch=800)(table, idx)
  return out.reshape(b, l, 1, d)


# speedup vs baseline: 3.5245x; 3.5245x over previous
import functools

import jax
import jax.numpy as jnp
from jax import lax
from jax.experimental import pallas as pl
from jax.experimental.pallas import tpu as pltpu
from jax.experimental.pallas import tpu_sc as plsc

_NUM_WORKERS = 32


def _make_gather(v, d, b, ch):
  bpw = b // _NUM_WORKERS
  nch = bpw // ch
  mesh = plsc.VectorSubcoreMesh(core_axis_name="c", subcore_axis_name="s")

  @functools.partial(
      pl.kernel,
      mesh=mesh,
      out_type=jax.ShapeDtypeStruct((b, d), jnp.float32),
      scratch_types=[
          pltpu.VMEM((ch,), jnp.int32),
          pltpu.VMEM((ch, d), jnp.float32),
          pltpu.SemaphoreType.DMA,
      ],
      compiler_params=pltpu.CompilerParams(use_tc_tiling_on_sc=False),
  )
  def gather(table_hbm, idx_hbm, out_hbm, idx_v, rows_v, sem):
    wid = lax.axis_index("s") * 2 + lax.axis_index("c")
    w_base = wid * bpw

    def body(j, carry):
      base = w_base + j * ch
      pltpu.sync_copy(idx_hbm.at[pl.ds(base, ch)], idx_v)
      pltpu.async_copy(table_hbm.at[idx_v], rows_v, sem).wait()
      pltpu.sync_copy(rows_v, out_hbm.at[pl.ds(base, ch)])
      return carry

    lax.fori_loop(0, nch, body, 0)

  return gather


def kernel(x, pe):
  b, l = x.shape
  v = pe.shape[0]
  d = pe.shape[-1]
  total = b * l
  idx = x.reshape(total)
  table = pe.reshape(v, d)
  out = _make_gather(v, d, total, ch=800)(table, idx)
  return out.reshape(b, l, 1, d)


# speedup vs baseline: 3.6352x; 1.0314x over previous
import functools

import jax
import jax.numpy as jnp
from jax import lax
from jax.experimental import pallas as pl
from jax.experimental.pallas import tpu as pltpu
from jax.experimental.pallas import tpu_sc as plsc

_NUM_WORKERS = 32


def _make_gather(v, d, b, ch):
  bpw = b // _NUM_WORKERS
  nch = bpw // ch
  mesh = plsc.VectorSubcoreMesh(core_axis_name="c", subcore_axis_name="s")

  @functools.partial(
      pl.kernel,
      mesh=mesh,
      out_type=jax.ShapeDtypeStruct((b, d), jnp.float32),
      scratch_types=[
          pltpu.VMEM((bpw,), jnp.int32),
          pltpu.VMEM((ch, d), jnp.float32),
          pltpu.VMEM((ch, d), jnp.float32),
          pltpu.SemaphoreType.DMA,
          pltpu.SemaphoreType.DMA,
          pltpu.SemaphoreType.DMA,
          pltpu.SemaphoreType.DMA,
      ],
      compiler_params=pltpu.CompilerParams(use_tc_tiling_on_sc=False),
  )
  def gather(table_hbm, idx_hbm, out_hbm, idx_v, rows0, rows1,
             gsem0, gsem1, ssem0, ssem1):
    wid = lax.axis_index("s") * 2 + lax.axis_index("c")
    w_base = wid * bpw
    rows = (rows0, rows1)
    gsem = (gsem0, gsem1)
    ssem = (ssem0, ssem1)

    pltpu.sync_copy(idx_hbm.at[pl.ds(w_base, bpw)], idx_v)

    ghandle = [None, None]
    shandle = [None, None]
    ghandle[0] = pltpu.async_copy(
        table_hbm.at[idx_v.at[pl.ds(0, ch)]], rows[0], gsem[0])
    for k in range(nch):
      cur = k % 2
      nxt = (k + 1) % 2
      if k + 1 < nch:
        if shandle[nxt] is not None:
          shandle[nxt].wait()
        ghandle[nxt] = pltpu.async_copy(
            table_hbm.at[idx_v.at[pl.ds((k + 1) * ch, ch)]], rows[nxt],
            gsem[nxt])
      ghandle[cur].wait()
      shandle[cur] = pltpu.async_copy(
          rows[cur], out_hbm.at[pl.ds(w_base + k * ch, ch)], ssem[cur])
    shandle[0].wait()
    shandle[1].wait()

  return gather


def kernel(x, pe):
  b, l = x.shape
  v = pe.shape[0]
  d = pe.shape[-1]
  total = b * l
  idx = x.reshape(total)
  table = pe.reshape(v, d)
  out = _make_gather(v, d, total, ch=800)(table, idx)
  return out.reshape(b, l, 1, d)
